# Initial kernel scaffold; baseline (speedup 1.0000x reference)
import jax
import jax.numpy as jnp
from jax.experimental import pallas as pl


def kernel(images, W):
    raise NotImplementedError("write your pallas kernel here")


# speedup vs baseline: 45.3168x; 45.3168x over previous
import functools

import jax
import jax.numpy as jnp
from jax import lax
from jax.experimental import pallas as pl
from jax.experimental.pallas import tpu as pltpu
from jax.experimental.pallas import tpu_sc as plsc

B = 8192
D_IN = 768
D = 256
K = 32
NSEL = K + 1

R = 256
CT = 512
T = 4
NC = 2
NS = 16
ROWS_PER_TILE = B // (NC * NS)

_BIG_I = 2**30
_INF = float("inf")


def _proj_body(img_ref, w_ref, out_ref):
    out_ref[...] = lax.dot_general(
        img_ref[...], w_ref[...], (((1,), (0,)), ((), ())),
        precision=lax.Precision.HIGHEST,
        preferred_element_type=jnp.float32)


def _project(images, w):
    rb = 512
    return pl.pallas_call(
        _proj_body,
        grid=(B // rb,),
        in_specs=[
            pl.BlockSpec((rb, D_IN), lambda i: (i, 0)),
            pl.BlockSpec((D_IN, D), lambda i: (0, 0)),
        ],
        out_specs=pl.BlockSpec((rb, D), lambda i: (i, 0)),
        out_shape=jax.ShapeDtypeStruct((B, D), jnp.float32),
    )(images, w)


def _topk_body(xr_ref, vf_ref, idx_ref, stats_ref,
               candv_ref, candi_ref, x2row_ref, vals_ref):
    i = pl.program_id(0)

    @pl.when(i == 0)
    def _():
        vf2 = vf_ref[...] * vf_ref[...]
        x2row_ref[...] = lax.dot_general(
            jnp.ones((8, D), jnp.float32), vf2, (((1,), (1,)), ((), ())),
            precision=lax.Precision.HIGHEST,
            preferred_element_type=jnp.float32)

    candv_ref[...] = jnp.full((R, T * CT), _INF, jnp.float32)
    candi_ref[...] = jnp.zeros((R, T * CT), jnp.int32)

    x_rb = xr_ref[...]
    x2_rb = jnp.sum(x_rb * x_rb, axis=1, keepdims=True)
    grow = (lax.broadcasted_iota(jnp.int32, (R, CT), 0) + i * R)

    def ct_body(ct, vd):
        vfc = vf_ref[pl.ds(ct * CT, CT), :]
        dots = lax.dot_general(
            x_rb, vfc, (((1,), (1,)), ((), ())),
            precision=lax.Precision.HIGHEST,
            preferred_element_type=jnp.float32)
        x2c = x2row_ref[0:1, pl.ds(ct * CT, CT)]
        d2 = x2_rb + x2c - 2.0 * dots
        gcol = lax.broadcasted_iota(jnp.int32, (R, CT), 1) + ct * CT
        diag = gcol == grow
        vd = vd + jnp.sum(jnp.where(diag, d2, 0.0), axis=1, keepdims=True)
        v_new = jnp.where(diag, _INF, d2)
        i_new = gcol
        for t in range(T):
            sl = pl.ds(t * CT, CT)
            cv = candv_ref[:, sl]
            ci = candi_ref[:, sl]
            swap = v_new < cv
            candv_ref[:, sl] = jnp.where(swap, v_new, cv)
            candi_ref[:, sl] = jnp.where(swap, i_new, ci)
            v_new = jnp.where(swap, cv, v_new)
            i_new = jnp.where(swap, ci, i_new)
        return vd

    v_d = lax.fori_loop(0, B // CT, ct_body,
                        jnp.zeros((R, 1), jnp.float32))

    lane128 = lax.broadcasted_iota(jnp.int32, (R, 128), 1)
    rowid = (lax.broadcasted_iota(jnp.int32, (R, 1), 0) + i * R)

    def ex_body(k, _):
        top_v = candv_ref[:, 0:CT]
        top_i = candi_ref[:, 0:CT]
        m = jnp.min(top_v, axis=1, keepdims=True)
        eq = top_v == m
        g = jnp.min(jnp.where(eq, top_i, _BIG_I), axis=1, keepdims=True)
        lane_mask = eq & (top_i == g)
        for t in range(T - 1):
            sl = pl.ds(t * CT, CT)
            sn = pl.ds((t + 1) * CT, CT)
            candv_ref[:, sl] = jnp.where(lane_mask, candv_ref[:, sn],
                                         candv_ref[:, sl])
            candi_ref[:, sl] = jnp.where(lane_mask, candi_ref[:, sn],
                                         candi_ref[:, sl])
        sl = pl.ds((T - 1) * CT, CT)
        candv_ref[:, sl] = jnp.where(lane_mask, _INF, candv_ref[:, sl])
        candi_ref[:, sl] = jnp.where(lane_mask, _BIG_I, candi_ref[:, sl])
        vals_ref[...] = jnp.where(lane128 == k, m, vals_ref[...])
        g_rm = g - (g > rowid).astype(jnp.int32)
        idx_ref[...] = jnp.where(lane128 == k, g_rm, idx_ref[...])
        return 0

    lax.fori_loop(0, NSEL, ex_body, 0)

    v31 = vals_ref[:, (K - 1):K]
    v32 = vals_ref[:, K:(K + 1)]
    a_k = jnp.sqrt(jnp.maximum(v32, 0.0))
    kth2 = jnp.minimum(jnp.maximum(v_d, v31), v32)
    kth = jnp.sqrt(jnp.maximum(kth2, 0.0))
    stats_ref[...] = jnp.where(
        lane128 == 0, a_k, jnp.where(lane128 == 1, kth, 0.0))


def _topk(vf):
    return pl.pallas_call(
        _topk_body,
        grid=(B // R,),
        in_specs=[
            pl.BlockSpec((R, D), lambda i: (i, 0)),
            pl.BlockSpec((B, D), lambda i: (0, 0)),
        ],
        out_specs=[
            pl.BlockSpec((R, 128), lambda i: (i, 0)),
            pl.BlockSpec((R, 128), lambda i: (i, 0)),
        ],
        out_shape=[
            jax.ShapeDtypeStruct((B, 128), jnp.int32),
            jax.ShapeDtypeStruct((B, 128), jnp.float32),
        ],
        scratch_shapes=[
            pltpu.VMEM((R, T * CT), jnp.float32),
            pltpu.VMEM((R, T * CT), jnp.int32),
            pltpu.VMEM((8, B), jnp.float32),
            pltpu.VMEM((R, 128), jnp.float32),
        ],
    )(vf, vf)


def _sc_score_body(kth_hbm, ak_hbm, idx_hbm, out_hbm,
                   kth_v, ak_v, idx_v, sc_v):
    wid = lax.axis_index("s") * NC + lax.axis_index("c")
    base = wid * ROWS_PER_TILE
    pltpu.sync_copy(kth_hbm, kth_v)
    pltpu.sync_copy(ak_hbm.at[pl.ds(base, ROWS_PER_TILE)], ak_v)
    pltpu.sync_copy(idx_hbm.at[wid], idx_v)

    def g_body(g, _):
        r0 = g * 16
        ak16 = ak_v[pl.ds(r0, 16)]
        acc = jnp.zeros((16,), jnp.float32)
        for m in range(K):
            iv = idx_v[pl.ds(m * ROWS_PER_TILE + r0, 16)]
            dk = plsc.load_gather(kth_v, [iv])
            acc = acc + ak16 / dk
        s = acc * (1.0 / K)
        s = jnp.where(s != s, 1000.0, s)
        s = jnp.where(s == _INF, 1000.0, s)
        s = jnp.where(s == -_INF, 0.0, s)
        sc_v[pl.ds(r0, 16)] = s
        return 0

    lax.fori_loop(0, ROWS_PER_TILE // 16, g_body, 0)
    pltpu.sync_copy(sc_v, out_hbm.at[pl.ds(base, ROWS_PER_TILE)])


@functools.cache
def _make_sc_score():
    return pl.kernel(
        _sc_score_body,
        out_type=jax.ShapeDtypeStruct((B,), jnp.float32),
        mesh=plsc.VectorSubcoreMesh(core_axis_name="c", subcore_axis_name="s",
                                    num_cores=NC, num_subcores=NS),
        compiler_params=pltpu.CompilerParams(needs_layout_passes=False),
        scratch_types=[
            pltpu.VMEM((B,), jnp.float32),
            pltpu.VMEM((ROWS_PER_TILE,), jnp.float32),
            pltpu.VMEM((K * ROWS_PER_TILE,), jnp.int32),
            pltpu.VMEM((ROWS_PER_TILE,), jnp.float32),
        ],
    )


def _score_stage(kth, a_k, idx_sc):
    return _make_sc_score()(kth, a_k, idx_sc)


def kernel(images, W):
    vf = _project(images, W)
    idx_out, stats_out = _topk(vf)
    a_k = stats_out[:, 0]
    kth = stats_out[:, 1]
    idx32 = idx_out[:, :K]
    idx_sc = (idx32.reshape(NC * NS, ROWS_PER_TILE, K)
              .transpose(0, 2, 1)
              .reshape(NC * NS, K * ROWS_PER_TILE))
    return _score_stage(kth, a_k, idx_sc)


# speedup vs baseline: 51.7651x; 1.1423x over previous
import functools

import jax
import jax.numpy as jnp
from jax import lax
from jax.experimental import pallas as pl
from jax.experimental.pallas import tpu as pltpu
from jax.experimental.pallas import tpu_sc as plsc

B = 8192
D_IN = 768
D = 256
K = 32
NSEL = K + 1

R = 256
CT = 512
T = 4
NC = 2
NS = 16
ROWS_PER_TILE = B // (NC * NS)

_BIG_I = 2**30
_INF = float("inf")


def _proj_body(img_ref, w_ref, out_ref):
    out_ref[...] = lax.dot_general(
        img_ref[...], w_ref[...], (((1,), (0,)), ((), ())),
        precision=lax.Precision.HIGHEST,
        preferred_element_type=jnp.float32)


def _project(images, w):
    rb = 512
    return pl.pallas_call(
        _proj_body,
        grid=(B // rb,),
        in_specs=[
            pl.BlockSpec((rb, D_IN), lambda i: (i, 0)),
            pl.BlockSpec((D_IN, D), lambda i: (0, 0)),
        ],
        out_specs=pl.BlockSpec((rb, D), lambda i: (i, 0)),
        out_shape=jax.ShapeDtypeStruct((B, D), jnp.float32),
    )(images, w)


def _topk_body(xr_ref, vf_ref, idx_ref, stats_ref,
               candv_ref, candi_ref, x2row_ref, vals_ref):
    i = pl.program_id(0)

    @pl.when(i == 0)
    def _():
        vff = vf_ref[...].astype(jnp.float32)
        vf2 = vff * vff
        x2row_ref[...] = lax.dot_general(
            jnp.ones((8, D), jnp.float32), vf2, (((1,), (1,)), ((), ())),
            precision=lax.Precision.HIGHEST,
            preferred_element_type=jnp.float32)

    candv_ref[...] = jnp.full((R, T * CT), _INF, jnp.float32)
    candi_ref[...] = jnp.zeros((R, T * CT), jnp.int32)

    x_rb = xr_ref[...]
    xf = x_rb.astype(jnp.float32)
    x2_rb = jnp.sum(xf * xf, axis=1, keepdims=True)
    grow = (lax.broadcasted_iota(jnp.int32, (R, CT), 0) + i * R)

    def ct_body(ct, vd):
        vfc = vf_ref[pl.ds(ct * CT, CT), :]
        dots = lax.dot_general(
            x_rb, vfc, (((1,), (1,)), ((), ())),
            preferred_element_type=jnp.float32)
        x2c = x2row_ref[0:1, pl.ds(ct * CT, CT)]
        d2 = x2_rb + x2c - 2.0 * dots
        gcol = lax.broadcasted_iota(jnp.int32, (R, CT), 1) + ct * CT
        diag = gcol == grow
        vd = vd + jnp.sum(jnp.where(diag, d2, 0.0), axis=1, keepdims=True)
        v_new = jnp.where(diag, _INF, d2)
        i_new = gcol
        for t in range(T):
            sl = pl.ds(t * CT, CT)
            cv = candv_ref[:, sl]
            ci = candi_ref[:, sl]
            swap = v_new < cv
            candv_ref[:, sl] = jnp.where(swap, v_new, cv)
            candi_ref[:, sl] = jnp.where(swap, i_new, ci)
            v_new = jnp.where(swap, cv, v_new)
            i_new = jnp.where(swap, ci, i_new)
        return vd

    v_d = lax.fori_loop(0, B // CT, ct_body,
                        jnp.zeros((R, 1), jnp.float32))

    lane128 = lax.broadcasted_iota(jnp.int32, (R, 128), 1)
    rowid = (lax.broadcasted_iota(jnp.int32, (R, 1), 0) + i * R)

    def ex_body(k, _):
        top_v = candv_ref[:, 0:CT]
        top_i = candi_ref[:, 0:CT]
        m = jnp.min(top_v, axis=1, keepdims=True)
        eq = top_v == m
        g = jnp.min(jnp.where(eq, top_i, _BIG_I), axis=1, keepdims=True)
        lane_mask = eq & (top_i == g)
        for t in range(T - 1):
            sl = pl.ds(t * CT, CT)
            sn = pl.ds((t + 1) * CT, CT)
            candv_ref[:, sl] = jnp.where(lane_mask, candv_ref[:, sn],
                                         candv_ref[:, sl])
            candi_ref[:, sl] = jnp.where(lane_mask, candi_ref[:, sn],
                                         candi_ref[:, sl])
        sl = pl.ds((T - 1) * CT, CT)
        candv_ref[:, sl] = jnp.where(lane_mask, _INF, candv_ref[:, sl])
        candi_ref[:, sl] = jnp.where(lane_mask, _BIG_I, candi_ref[:, sl])
        vals_ref[...] = jnp.where(lane128 == k, m, vals_ref[...])
        g_rm = g - (g > rowid).astype(jnp.int32)
        idx_ref[...] = jnp.where(lane128 == k, g_rm, idx_ref[...])
        return 0

    lax.fori_loop(0, NSEL, ex_body, 0)

    v31 = vals_ref[:, (K - 1):K]
    v32 = vals_ref[:, K:(K + 1)]
    a_k = jnp.sqrt(jnp.maximum(v32, 0.0))
    kth2 = jnp.minimum(jnp.maximum(v_d, v31), v32)
    kth = jnp.sqrt(jnp.maximum(kth2, 0.0))
    stats_ref[...] = jnp.where(
        lane128 == 0, a_k, jnp.where(lane128 == 1, kth, 0.0))


def _topk(vf):
    vfb = vf.astype(jnp.bfloat16)
    return pl.pallas_call(
        _topk_body,
        grid=(B // R,),
        in_specs=[
            pl.BlockSpec((R, D), lambda i: (i, 0)),
            pl.BlockSpec((B, D), lambda i: (0, 0)),
        ],
        out_specs=[
            pl.BlockSpec((R, 128), lambda i: (i, 0)),
            pl.BlockSpec((R, 128), lambda i: (i, 0)),
        ],
        out_shape=[
            jax.ShapeDtypeStruct((B, 128), jnp.int32),
            jax.ShapeDtypeStruct((B, 128), jnp.float32),
        ],
        scratch_shapes=[
            pltpu.VMEM((R, T * CT), jnp.float32),
            pltpu.VMEM((R, T * CT), jnp.int32),
            pltpu.VMEM((8, B), jnp.float32),
            pltpu.VMEM((R, 128), jnp.float32),
        ],
    )(vfb, vfb)


def _sc_score_body(kth_hbm, ak_hbm, idx_hbm, out_hbm,
                   kth_v, ak_v, idx_v, sc_v):
    wid = lax.axis_index("s") * NC + lax.axis_index("c")
    base = wid * ROWS_PER_TILE
    pltpu.sync_copy(kth_hbm, kth_v)
    pltpu.sync_copy(ak_hbm.at[pl.ds(base, ROWS_PER_TILE)], ak_v)
    pltpu.sync_copy(idx_hbm.at[wid], idx_v)

    def g_body(g, _):
        r0 = g * 16
        ak16 = ak_v[pl.ds(r0, 16)]
        acc = jnp.zeros((16,), jnp.float32)
        for m in range(K):
            iv = idx_v[pl.ds(m * ROWS_PER_TILE + r0, 16)]
            dk = plsc.load_gather(kth_v, [iv])
            acc = acc + ak16 / dk
        s = acc * (1.0 / K)
        s = jnp.where(s != s, 1000.0, s)
        s = jnp.where(s == _INF, 1000.0, s)
        s = jnp.where(s == -_INF, 0.0, s)
        sc_v[pl.ds(r0, 16)] = s
        return 0

    lax.fori_loop(0, ROWS_PER_TILE // 16, g_body, 0)
    pltpu.sync_copy(sc_v, out_hbm.at[pl.ds(base, ROWS_PER_TILE)])


@functools.cache
def _make_sc_score():
    return pl.kernel(
        _sc_score_body,
        out_type=jax.ShapeDtypeStruct((B,), jnp.float32),
        mesh=plsc.VectorSubcoreMesh(core_axis_name="c", subcore_axis_name="s",
                                    num_cores=NC, num_subcores=NS),
        compiler_params=pltpu.CompilerParams(needs_layout_passes=False),
        scratch_types=[
            pltpu.VMEM((B,), jnp.float32),
            pltpu.VMEM((ROWS_PER_TILE,), jnp.float32),
            pltpu.VMEM((K * ROWS_PER_TILE,), jnp.int32),
            pltpu.VMEM((ROWS_PER_TILE,), jnp.float32),
        ],
    )


def _score_stage(kth, a_k, idx_sc):
    return _make_sc_score()(kth, a_k, idx_sc)


def kernel(images, W):
    vf = _project(images, W)
    idx_out, stats_out = _topk(vf)
    a_k = stats_out[:, 0]
    kth = stats_out[:, 1]
    idx32 = idx_out[:, :K]
    idx_sc = (idx32.reshape(NC * NS, ROWS_PER_TILE, K)
              .transpose(0, 2, 1)
              .reshape(NC * NS, K * ROWS_PER_TILE))
    return _score_stage(kth, a_k, idx_sc)


# speedup vs baseline: 57.6637x; 1.1139x over previous
import functools

import jax
import jax.numpy as jnp
from jax import lax
from jax.experimental import pallas as pl
from jax.experimental.pallas import tpu as pltpu
from jax.experimental.pallas import tpu_sc as plsc

B = 8192
D_IN = 768
D = 256
K = 32
NSEL = K + 1

R = 256
CT = 512
T = 4
NC = 2
NS = 16
ROWS_PER_TILE = B // (NC * NS)

_BIG_I = 2**30
_INF = float("inf")


def _proj_body(img_ref, w_ref, out_ref):
    out_ref[...] = lax.dot_general(
        img_ref[...], w_ref[...], (((1,), (0,)), ((), ())),
        precision=lax.Precision.HIGHEST,
        preferred_element_type=jnp.float32)


def _project(images, w):
    rb = 512
    return pl.pallas_call(
        _proj_body,
        grid=(B // rb,),
        in_specs=[
            pl.BlockSpec((rb, D_IN), lambda i: (i, 0)),
            pl.BlockSpec((D_IN, D), lambda i: (0, 0)),
        ],
        out_specs=pl.BlockSpec((rb, D), lambda i: (i, 0)),
        out_shape=jax.ShapeDtypeStruct((B, D), jnp.float32),
    )(images, w)


def _topk_body(xr_ref, vf_ref, idx_ref, stats_ref,
               candv_ref, candi_ref, x2row_ref, vals_ref):
    i = pl.program_id(0)

    @pl.when(i == 0)
    def _():
        vff = vf_ref[...].astype(jnp.float32)
        vf2 = vff * vff
        x2row_ref[...] = lax.dot_general(
            jnp.ones((8, D), jnp.float32), vf2, (((1,), (1,)), ((), ())),
            precision=lax.Precision.HIGHEST,
            preferred_element_type=jnp.float32)

    candv_ref[...] = jnp.full((R, T * CT), _INF, jnp.float32)
    candi_ref[...] = jnp.zeros((R, T * CT), jnp.int32)

    x_rb = xr_ref[...]
    xf = x_rb.astype(jnp.float32)
    x2_rb = jnp.sum(xf * xf, axis=1, keepdims=True)
    grow = (lax.broadcasted_iota(jnp.int32, (R, CT), 0) + i * R)

    def ct_body(ct, vd):
        vfc = vf_ref[pl.ds(ct * CT, CT), :]
        dots = lax.dot_general(
            x_rb, vfc, (((1,), (1,)), ((), ())),
            preferred_element_type=jnp.float32)
        x2c = x2row_ref[0:1, pl.ds(ct * CT, CT)]
        d2 = x2_rb + x2c - 2.0 * dots
        gcol = lax.broadcasted_iota(jnp.int32, (R, CT), 1) + ct * CT
        diag = gcol == grow
        vd = vd + jnp.sum(jnp.where(diag, d2, 0.0), axis=1, keepdims=True)
        v_new = jnp.where(diag, _INF, d2)
        i_new = gcol
        for t in range(T):
            sl = pl.ds(t * CT, CT)
            cv = candv_ref[:, sl]
            ci = candi_ref[:, sl]
            swap = v_new < cv
            candv_ref[:, sl] = jnp.where(swap, v_new, cv)
            candi_ref[:, sl] = jnp.where(swap, i_new, ci)
            v_new = jnp.where(swap, cv, v_new)
            i_new = jnp.where(swap, ci, i_new)
        return vd

    v_d = lax.fori_loop(0, B // CT, ct_body,
                        jnp.zeros((R, 1), jnp.float32))

    def _mmin(a, b):
        sw = b[0] < a[0]
        return (jnp.where(sw, b[0], a[0]), jnp.where(sw, b[1], a[1]))

    def _ce(a, b):
        sw = b[0] < a[0]
        return ((jnp.where(sw, b[0], a[0]), jnp.where(sw, b[1], a[1])),
                (jnp.where(sw, a[0], b[0]), jnp.where(sw, a[1], b[1])))

    def _merge2(A, Bl):
        lo = [_mmin(A[k], Bl[T - 1 - k]) for k in range(T)]
        lo[0], lo[2] = _ce(lo[0], lo[2])
        lo[1], lo[3] = _ce(lo[1], lo[3])
        lo[0], lo[1] = _ce(lo[0], lo[1])
        lo[2], lo[3] = _ce(lo[2], lo[3])
        return lo

    cols = []
    for c in range(4):
        cols.append([(candv_ref[:, t * CT + c * 128: t * CT + c * 128 + 128],
                      candi_ref[:, t * CT + c * 128: t * CT + c * 128 + 128])
                     for t in range(T)])
    merged = _merge2(_merge2(cols[0], cols[1]), _merge2(cols[2], cols[3]))
    for t in range(T):
        candv_ref[:, t * CT: t * CT + 128] = merged[t][0]
        candi_ref[:, t * CT: t * CT + 128] = merged[t][1]

    lane128 = lax.broadcasted_iota(jnp.int32, (R, 128), 1)
    rowid = (lax.broadcasted_iota(jnp.int32, (R, 1), 0) + i * R)

    def ex_body(k, _):
        top_v = candv_ref[:, 0:128]
        top_i = candi_ref[:, 0:128]
        m = jnp.min(top_v, axis=1, keepdims=True)
        eq = top_v == m
        g = jnp.min(jnp.where(eq, top_i, _BIG_I), axis=1, keepdims=True)
        lane_mask = eq & (top_i == g)
        for t in range(T - 1):
            sl = pl.ds(t * CT, 128)
            sn = pl.ds((t + 1) * CT, 128)
            candv_ref[:, sl] = jnp.where(lane_mask, candv_ref[:, sn],
                                         candv_ref[:, sl])
            candi_ref[:, sl] = jnp.where(lane_mask, candi_ref[:, sn],
                                         candi_ref[:, sl])
        sl = pl.ds((T - 1) * CT, 128)
        candv_ref[:, sl] = jnp.where(lane_mask, _INF, candv_ref[:, sl])
        candi_ref[:, sl] = jnp.where(lane_mask, _BIG_I, candi_ref[:, sl])
        vals_ref[...] = jnp.where(lane128 == k, m, vals_ref[...])
        g_rm = g - (g > rowid).astype(jnp.int32)
        idx_ref[...] = jnp.where(lane128 == k, g_rm, idx_ref[...])
        return 0

    lax.fori_loop(0, NSEL, ex_body, 0)

    v31 = vals_ref[:, (K - 1):K]
    v32 = vals_ref[:, K:(K + 1)]
    a_k = jnp.sqrt(jnp.maximum(v32, 0.0))
    kth2 = jnp.minimum(jnp.maximum(v_d, v31), v32)
    kth = jnp.sqrt(jnp.maximum(kth2, 0.0))
    stats_ref[...] = jnp.where(
        lane128 == 0, a_k, jnp.where(lane128 == 1, kth, 0.0))


def _topk(vf):
    vfb = vf.astype(jnp.bfloat16)
    return pl.pallas_call(
        _topk_body,
        grid=(B // R,),
        in_specs=[
            pl.BlockSpec((R, D), lambda i: (i, 0)),
            pl.BlockSpec((B, D), lambda i: (0, 0)),
        ],
        out_specs=[
            pl.BlockSpec((R, 128), lambda i: (i, 0)),
            pl.BlockSpec((R, 128), lambda i: (i, 0)),
        ],
        out_shape=[
            jax.ShapeDtypeStruct((B, 128), jnp.int32),
            jax.ShapeDtypeStruct((B, 128), jnp.float32),
        ],
        scratch_shapes=[
            pltpu.VMEM((R, T * CT), jnp.float32),
            pltpu.VMEM((R, T * CT), jnp.int32),
            pltpu.VMEM((8, B), jnp.float32),
            pltpu.VMEM((R, 128), jnp.float32),
        ],
    )(vfb, vfb)


def _sc_score_body(kth_hbm, ak_hbm, idx_hbm, out_hbm,
                   kth_v, ak_v, idx_v, sc_v):
    wid = lax.axis_index("s") * NC + lax.axis_index("c")
    base = wid * ROWS_PER_TILE
    pltpu.sync_copy(kth_hbm, kth_v)
    pltpu.sync_copy(ak_hbm.at[pl.ds(base, ROWS_PER_TILE)], ak_v)
    pltpu.sync_copy(idx_hbm.at[wid], idx_v)

    def g_body(g, _):
        r0 = g * 16
        ak16 = ak_v[pl.ds(r0, 16)]
        acc = jnp.zeros((16,), jnp.float32)
        for m in range(K):
            iv = idx_v[pl.ds(m * ROWS_PER_TILE + r0, 16)]
            dk = plsc.load_gather(kth_v, [iv])
            acc = acc + ak16 / dk
        s = acc * (1.0 / K)
        s = jnp.where(s != s, 1000.0, s)
        s = jnp.where(s == _INF, 1000.0, s)
        s = jnp.where(s == -_INF, 0.0, s)
        sc_v[pl.ds(r0, 16)] = s
        return 0

    lax.fori_loop(0, ROWS_PER_TILE // 16, g_body, 0)
    pltpu.sync_copy(sc_v, out_hbm.at[pl.ds(base, ROWS_PER_TILE)])


@functools.cache
def _make_sc_score():
    return pl.kernel(
        _sc_score_body,
        out_type=jax.ShapeDtypeStruct((B,), jnp.float32),
        mesh=plsc.VectorSubcoreMesh(core_axis_name="c", subcore_axis_name="s",
                                    num_cores=NC, num_subcores=NS),
        compiler_params=pltpu.CompilerParams(needs_layout_passes=False),
        scratch_types=[
            pltpu.VMEM((B,), jnp.float32),
            pltpu.VMEM((ROWS_PER_TILE,), jnp.float32),
            pltpu.VMEM((K * ROWS_PER_TILE,), jnp.int32),
            pltpu.VMEM((ROWS_PER_TILE,), jnp.float32),
        ],
    )


def _score_stage(kth, a_k, idx_sc):
    return _make_sc_score()(kth, a_k, idx_sc)


def kernel(images, W):
    vf = _project(images, W)
    idx_out, stats_out = _topk(vf)
    a_k = stats_out[:, 0]
    kth = stats_out[:, 1]
    idx32 = idx_out[:, :K]
    idx_sc = (idx32.reshape(NC * NS, ROWS_PER_TILE, K)
              .transpose(0, 2, 1)
              .reshape(NC * NS, K * ROWS_PER_TILE))
    return _score_stage(kth, a_k, idx_sc)


# speedup vs baseline: 70.4149x; 1.2211x over previous
import functools

import jax
import jax.numpy as jnp
from jax import lax
from jax.experimental import pallas as pl
from jax.experimental.pallas import tpu as pltpu
from jax.experimental.pallas import tpu_sc as plsc

B = 8192
D_IN = 768
D = 256
K = 32
NSEL = K + 1

R = 256
CT = 512
CTB = 2048
T = 4
NC = 2
NS = 16
ROWS_PER_TILE = B // (NC * NS)

_BIG_I = 2**30
_INF = float("inf")


def _proj_body(img_ref, w_ref, out_ref):
    out_ref[...] = lax.dot_general(
        img_ref[...], w_ref[...], (((1,), (0,)), ((), ())),
        precision=lax.Precision.HIGHEST,
        preferred_element_type=jnp.float32)


def _project(images, w):
    rb = 512
    return pl.pallas_call(
        _proj_body,
        grid=(B // rb,),
        in_specs=[
            pl.BlockSpec((rb, D_IN), lambda i: (i, 0)),
            pl.BlockSpec((D_IN, D), lambda i: (0, 0)),
        ],
        out_specs=pl.BlockSpec((rb, D), lambda i: (i, 0)),
        out_shape=jax.ShapeDtypeStruct((B, D), jnp.float32),
    )(images, w)


def _topk_body(xr_ref, vf_ref, idx_ref, stats_ref,
               candv_ref, candi_ref, x2row_ref, vals_ref):
    i = pl.program_id(0)

    @pl.when(i == 0)
    def _():
        vff = vf_ref[...].astype(jnp.float32)
        vf2 = vff * vff
        x2row_ref[...] = lax.dot_general(
            jnp.ones((8, D), jnp.float32), vf2, (((1,), (1,)), ((), ())),
            precision=lax.Precision.HIGHEST,
            preferred_element_type=jnp.float32)

    candv_ref[...] = jnp.full((R, T * CT), _INF, jnp.float32)
    candi_ref[...] = jnp.zeros((R, T * CT), jnp.int32)

    x_rb = xr_ref[...]
    xf = x_rb.astype(jnp.float32)
    x2_rb = jnp.sum(xf * xf, axis=1, keepdims=True)
    grow = (lax.broadcasted_iota(jnp.int32, (R, CTB), 0) + i * R)

    def _mmin(a, b):
        sw = b[0] < a[0]
        return (jnp.where(sw, b[0], a[0]), jnp.where(sw, b[1], a[1]))

    def _ce(a, b):
        sw = b[0] < a[0]
        return ((jnp.where(sw, b[0], a[0]), jnp.where(sw, b[1], a[1])),
                (jnp.where(sw, a[0], b[0]), jnp.where(sw, a[1], b[1])))

    def _sort4(s):
        s[0], s[1] = _ce(s[0], s[1])
        s[2], s[3] = _ce(s[2], s[3])
        s[0], s[2] = _ce(s[0], s[2])
        s[1], s[3] = _ce(s[1], s[3])
        s[1], s[2] = _ce(s[1], s[2])
        return s

    def _merge2(A, Bl):
        lo = [_mmin(A[k], Bl[T - 1 - k]) for k in range(T)]
        lo[0], lo[2] = _ce(lo[0], lo[2])
        lo[1], lo[3] = _ce(lo[1], lo[3])
        lo[0], lo[1] = _ce(lo[0], lo[1])
        lo[2], lo[3] = _ce(lo[2], lo[3])
        return lo

    def ct_body(ct, vd):
        vfc = vf_ref[pl.ds(ct * CTB, CTB), :]
        dots = lax.dot_general(
            x_rb, vfc, (((1,), (1,)), ((), ())),
            preferred_element_type=jnp.float32)
        x2c = x2row_ref[0:1, pl.ds(ct * CTB, CTB)]
        d2 = x2_rb + x2c - 2.0 * dots
        gcol = lax.broadcasted_iota(jnp.int32, (R, CTB), 1) + ct * CTB
        diag = gcol == grow
        vd = vd + jnp.sum(jnp.where(diag, d2, 0.0), axis=1, keepdims=True)
        v_new = jnp.where(diag, _INF, d2)
        s = [(v_new[:, q * CT:(q + 1) * CT], gcol[:, q * CT:(q + 1) * CT])
             for q in range(4)]
        s = _sort4(s)
        lists = [(candv_ref[:, t * CT:(t + 1) * CT],
                  candi_ref[:, t * CT:(t + 1) * CT]) for t in range(T)]
        merged = _merge2(lists, s)
        for t in range(T):
            candv_ref[:, t * CT:(t + 1) * CT] = merged[t][0]
            candi_ref[:, t * CT:(t + 1) * CT] = merged[t][1]
        return vd

    v_d = lax.fori_loop(0, B // CTB, ct_body,
                        jnp.zeros((R, 1), jnp.float32))

    cols = []
    for c in range(4):
        cols.append([(candv_ref[:, t * CT + c * 128: t * CT + c * 128 + 128],
                      candi_ref[:, t * CT + c * 128: t * CT + c * 128 + 128])
                     for t in range(T)])
    merged = _merge2(_merge2(cols[0], cols[1]), _merge2(cols[2], cols[3]))
    for t in range(T):
        candv_ref[:, t * CT: t * CT + 128] = merged[t][0]
        candi_ref[:, t * CT: t * CT + 128] = merged[t][1]

    lane128 = lax.broadcasted_iota(jnp.int32, (R, 128), 1)
    rowid = (lax.broadcasted_iota(jnp.int32, (R, 1), 0) + i * R)

    def ex_body(k, _):
        top_v = candv_ref[:, 0:128]
        top_i = candi_ref[:, 0:128]
        m = jnp.min(top_v, axis=1, keepdims=True)
        eq = top_v == m
        g = jnp.min(jnp.where(eq, top_i, _BIG_I), axis=1, keepdims=True)
        lane_mask = eq & (top_i == g)
        for t in range(T - 1):
            sl = pl.ds(t * CT, 128)
            sn = pl.ds((t + 1) * CT, 128)
            candv_ref[:, sl] = jnp.where(lane_mask, candv_ref[:, sn],
                                         candv_ref[:, sl])
            candi_ref[:, sl] = jnp.where(lane_mask, candi_ref[:, sn],
                                         candi_ref[:, sl])
        sl = pl.ds((T - 1) * CT, 128)
        candv_ref[:, sl] = jnp.where(lane_mask, _INF, candv_ref[:, sl])
        candi_ref[:, sl] = jnp.where(lane_mask, _BIG_I, candi_ref[:, sl])
        vals_ref[...] = jnp.where(lane128 == k, m, vals_ref[...])
        g_rm = g - (g > rowid).astype(jnp.int32)
        idx_ref[...] = jnp.where(lane128 == k, g_rm, idx_ref[...])
        return 0

    lax.fori_loop(0, NSEL, ex_body, 0)

    v31 = vals_ref[:, (K - 1):K]
    v32 = vals_ref[:, K:(K + 1)]
    a_k = jnp.sqrt(jnp.maximum(v32, 0.0))
    kth2 = jnp.minimum(jnp.maximum(v_d, v31), v32)
    kth = jnp.sqrt(jnp.maximum(kth2, 0.0))
    stats_ref[...] = jnp.where(
        lane128 == 0, a_k, jnp.where(lane128 == 1, kth, 0.0))


def _topk(vf):
    vfb = vf.astype(jnp.bfloat16)
    return pl.pallas_call(
        _topk_body,
        grid=(B // R,),
        in_specs=[
            pl.BlockSpec((R, D), lambda i: (i, 0)),
            pl.BlockSpec((B, D), lambda i: (0, 0)),
        ],
        out_specs=[
            pl.BlockSpec((R, 128), lambda i: (i, 0)),
            pl.BlockSpec((R, 128), lambda i: (i, 0)),
        ],
        out_shape=[
            jax.ShapeDtypeStruct((B, 128), jnp.int32),
            jax.ShapeDtypeStruct((B, 128), jnp.float32),
        ],
        scratch_shapes=[
            pltpu.VMEM((R, T * CT), jnp.float32),
            pltpu.VMEM((R, T * CT), jnp.int32),
            pltpu.VMEM((8, B), jnp.float32),
            pltpu.VMEM((R, 128), jnp.float32),
        ],
    )(vfb, vfb)


def _sc_score_body(kth_hbm, ak_hbm, idx_hbm, out_hbm,
                   kth_v, ak_v, idx_v, sc_v):
    wid = lax.axis_index("s") * NC + lax.axis_index("c")
    base = wid * ROWS_PER_TILE
    pltpu.sync_copy(kth_hbm, kth_v)
    pltpu.sync_copy(ak_hbm.at[pl.ds(base, ROWS_PER_TILE)], ak_v)
    pltpu.sync_copy(idx_hbm.at[wid], idx_v)

    def g_body(g, _):
        r0 = g * 16
        ak16 = ak_v[pl.ds(r0, 16)]
        acc = jnp.zeros((16,), jnp.float32)
        for m in range(K):
            iv = idx_v[pl.ds(m * ROWS_PER_TILE + r0, 16)]
            dk = plsc.load_gather(kth_v, [iv])
            acc = acc + ak16 / dk
        s = acc * (1.0 / K)
        s = jnp.where(s != s, 1000.0, s)
        s = jnp.where(s == _INF, 1000.0, s)
        s = jnp.where(s == -_INF, 0.0, s)
        sc_v[pl.ds(r0, 16)] = s
        return 0

    lax.fori_loop(0, ROWS_PER_TILE // 16, g_body, 0)
    pltpu.sync_copy(sc_v, out_hbm.at[pl.ds(base, ROWS_PER_TILE)])


@functools.cache
def _make_sc_score():
    return pl.kernel(
        _sc_score_body,
        out_type=jax.ShapeDtypeStruct((B,), jnp.float32),
        mesh=plsc.VectorSubcoreMesh(core_axis_name="c", subcore_axis_name="s",
                                    num_cores=NC, num_subcores=NS),
        compiler_params=pltpu.CompilerParams(needs_layout_passes=False),
        scratch_types=[
            pltpu.VMEM((B,), jnp.float32),
            pltpu.VMEM((ROWS_PER_TILE,), jnp.float32),
            pltpu.VMEM((K * ROWS_PER_TILE,), jnp.int32),
            pltpu.VMEM((ROWS_PER_TILE,), jnp.float32),
        ],
    )


def _score_stage(kth, a_k, idx_sc):
    return _make_sc_score()(kth, a_k, idx_sc)


def kernel(images, W):
    vf = _project(images, W)
    idx_out, stats_out = _topk(vf)
    a_k = stats_out[:, 0]
    kth = stats_out[:, 1]
    idx32 = idx_out[:, :K]
    idx_sc = (idx32.reshape(NC * NS, ROWS_PER_TILE, K)
              .transpose(0, 2, 1)
              .reshape(NC * NS, K * ROWS_PER_TILE))
    return _score_stage(kth, a_k, idx_sc)


# speedup vs baseline: 98.6105x; 1.4004x over previous
import functools

import jax
import jax.numpy as jnp
from jax import lax
from jax.experimental import pallas as pl
from jax.experimental.pallas import tpu as pltpu
from jax.experimental.pallas import tpu_sc as plsc

B = 8192
D_IN = 768
D = 256
K = 32
NSEL = K + 1

R = 256
CT = 512
CTB = 2048
T = 4
NC = 2
NS = 16
ROWS_PER_TILE = B // (NC * NS)

_BIG_I = 2**31 - 1
_INF = float("inf")


def _proj_body(img_ref, w_ref, out_ref):
    out_ref[...] = lax.dot_general(
        img_ref[...], w_ref[...], (((1,), (0,)), ((), ())),
        precision=lax.Precision.HIGHEST,
        preferred_element_type=jnp.float32)


def _project(images, w):
    rb = 512
    return pl.pallas_call(
        _proj_body,
        grid=(B // rb,),
        in_specs=[
            pl.BlockSpec((rb, D_IN), lambda i: (i, 0)),
            pl.BlockSpec((D_IN, D), lambda i: (0, 0)),
        ],
        out_specs=pl.BlockSpec((rb, D), lambda i: (i, 0)),
        out_shape=jax.ShapeDtypeStruct((B, D), jnp.float32),
    )(images, w)


def _topk_body(xr_ref, vf_ref, idx_ref, stats_ref,
               candk_ref, x2row_ref, vals_ref):
    i = pl.program_id(0)

    @pl.when(i == 0)
    def _():
        vff = vf_ref[...].astype(jnp.float32)
        vf2 = vff * vff
        x2row_ref[...] = lax.dot_general(
            jnp.ones((8, D), jnp.float32), vf2, (((1,), (1,)), ((), ())),
            precision=lax.Precision.HIGHEST,
            preferred_element_type=jnp.float32)

    candk_ref[...] = jnp.full((R, T * CT), _BIG_I, jnp.int32)

    x_rb = xr_ref[...]
    xf = x_rb.astype(jnp.float32)
    x2_rb = jnp.sum(xf * xf, axis=1, keepdims=True)
    grow = (lax.broadcasted_iota(jnp.int32, (R, CTB), 0) + i * R)

    def _ce(a, b):
        return jnp.minimum(a, b), jnp.maximum(a, b)

    def _sort4(s):
        s[0], s[1] = _ce(s[0], s[1])
        s[2], s[3] = _ce(s[2], s[3])
        s[0], s[2] = _ce(s[0], s[2])
        s[1], s[3] = _ce(s[1], s[3])
        s[1], s[2] = _ce(s[1], s[2])
        return s

    def _merge2(A, Bl):
        lo = [jnp.minimum(A[k], Bl[T - 1 - k]) for k in range(T)]
        lo[0], lo[2] = _ce(lo[0], lo[2])
        lo[1], lo[3] = _ce(lo[1], lo[3])
        lo[0], lo[1] = _ce(lo[0], lo[1])
        lo[2], lo[3] = _ce(lo[2], lo[3])
        return lo

    def ct_body(ct, vd):
        vfc = vf_ref[pl.ds(ct * CTB, CTB), :]
        dots = lax.dot_general(
            x_rb, vfc, (((1,), (1,)), ((), ())),
            preferred_element_type=jnp.float32)
        x2c = x2row_ref[0:1, pl.ds(ct * CTB, CTB)]
        d2 = x2_rb + x2c - 2.0 * dots
        gcol = lax.broadcasted_iota(jnp.int32, (R, CTB), 1) + ct * CTB
        diag = gcol == grow
        vd = vd + jnp.sum(jnp.where(diag, d2, 0.0), axis=1, keepdims=True)
        bits = lax.bitcast_convert_type(jnp.maximum(d2, 0.0), jnp.int32)
        keys = (bits & ~0x1FFF) | gcol
        keys = jnp.where(diag, _BIG_I, keys)
        s = _sort4([keys[:, q * CT:(q + 1) * CT] for q in range(4)])
        lists = [candk_ref[:, t * CT:(t + 1) * CT] for t in range(T)]
        merged = _merge2(lists, s)
        for t in range(T):
            candk_ref[:, t * CT:(t + 1) * CT] = merged[t]
        return vd

    v_d = lax.fori_loop(0, B // CTB, ct_body,
                        jnp.zeros((R, 1), jnp.float32))

    cols = []
    for c in range(4):
        cols.append([candk_ref[:, t * CT + c * 128: t * CT + c * 128 + 128]
                     for t in range(T)])
    merged = _merge2(_merge2(cols[0], cols[1]), _merge2(cols[2], cols[3]))
    for t in range(T):
        candk_ref[:, t * CT: t * CT + 128] = merged[t]

    lane128 = lax.broadcasted_iota(jnp.int32, (R, 128), 1)
    rowid = (lax.broadcasted_iota(jnp.int32, (R, 1), 0) + i * R)

    def ex_body(k, _):
        top = candk_ref[:, 0:128]
        m = jnp.min(top, axis=1, keepdims=True)
        lane_mask = top == m
        for t in range(T - 1):
            sl = pl.ds(t * CT, 128)
            sn = pl.ds((t + 1) * CT, 128)
            candk_ref[:, sl] = jnp.where(lane_mask, candk_ref[:, sn],
                                         candk_ref[:, sl])
        sl = pl.ds((T - 1) * CT, 128)
        candk_ref[:, sl] = jnp.where(lane_mask, _BIG_I, candk_ref[:, sl])
        val = lax.bitcast_convert_type(m & ~0x1FFF, jnp.float32)
        vals_ref[...] = jnp.where(lane128 == k, val, vals_ref[...])
        g = m & 0x1FFF
        g_rm = g - (g > rowid).astype(jnp.int32)
        idx_ref[...] = jnp.where(lane128 == k, g_rm, idx_ref[...])
        return 0

    lax.fori_loop(0, NSEL, ex_body, 0)

    v31 = vals_ref[:, (K - 1):K]
    v32 = vals_ref[:, K:(K + 1)]
    a_k = jnp.sqrt(jnp.maximum(v32, 0.0))
    kth2 = jnp.minimum(jnp.maximum(v_d, v31), v32)
    kth = jnp.sqrt(jnp.maximum(kth2, 0.0))
    stats_ref[...] = jnp.where(
        lane128 == 0, a_k, jnp.where(lane128 == 1, kth, 0.0))


def _topk(vf):
    vfb = vf.astype(jnp.bfloat16)
    return pl.pallas_call(
        _topk_body,
        grid=(B // R,),
        in_specs=[
            pl.BlockSpec((R, D), lambda i: (i, 0)),
            pl.BlockSpec((B, D), lambda i: (0, 0)),
        ],
        out_specs=[
            pl.BlockSpec((R, 128), lambda i: (i, 0)),
            pl.BlockSpec((R, 128), lambda i: (i, 0)),
        ],
        out_shape=[
            jax.ShapeDtypeStruct((B, 128), jnp.int32),
            jax.ShapeDtypeStruct((B, 128), jnp.float32),
        ],
        scratch_shapes=[
            pltpu.VMEM((R, T * CT), jnp.int32),
            pltpu.VMEM((8, B), jnp.float32),
            pltpu.VMEM((R, 128), jnp.float32),
        ],
    )(vfb, vfb)


def _sc_score_body(kth_hbm, ak_hbm, idx_hbm, out_hbm,
                   kth_v, ak_v, idx_v, sc_v):
    wid = lax.axis_index("s") * NC + lax.axis_index("c")
    base = wid * ROWS_PER_TILE
    pltpu.sync_copy(kth_hbm, kth_v)
    pltpu.sync_copy(ak_hbm.at[pl.ds(base, ROWS_PER_TILE)], ak_v)
    pltpu.sync_copy(idx_hbm.at[wid], idx_v)

    def g_body(g, _):
        r0 = g * 16
        ak16 = ak_v[pl.ds(r0, 16)]
        acc = jnp.zeros((16,), jnp.float32)
        for m in range(K):
            iv = idx_v[pl.ds(m * ROWS_PER_TILE + r0, 16)]
            dk = plsc.load_gather(kth_v, [iv])
            acc = acc + ak16 / dk
        s = acc * (1.0 / K)
        s = jnp.where(s != s, 1000.0, s)
        s = jnp.where(s == _INF, 1000.0, s)
        s = jnp.where(s == -_INF, 0.0, s)
        sc_v[pl.ds(r0, 16)] = s
        return 0

    lax.fori_loop(0, ROWS_PER_TILE // 16, g_body, 0)
    pltpu.sync_copy(sc_v, out_hbm.at[pl.ds(base, ROWS_PER_TILE)])


@functools.cache
def _make_sc_score():
    return pl.kernel(
        _sc_score_body,
        out_type=jax.ShapeDtypeStruct((B,), jnp.float32),
        mesh=plsc.VectorSubcoreMesh(core_axis_name="c", subcore_axis_name="s",
                                    num_cores=NC, num_subcores=NS),
        compiler_params=pltpu.CompilerParams(needs_layout_passes=False),
        scratch_types=[
            pltpu.VMEM((B,), jnp.float32),
            pltpu.VMEM((ROWS_PER_TILE,), jnp.float32),
            pltpu.VMEM((K * ROWS_PER_TILE,), jnp.int32),
            pltpu.VMEM((ROWS_PER_TILE,), jnp.float32),
        ],
    )


def _score_stage(kth, a_k, idx_sc):
    return _make_sc_score()(kth, a_k, idx_sc)


def kernel(images, W):
    vf = _project(images, W)
    idx_out, stats_out = _topk(vf)
    a_k = stats_out[:, 0]
    kth = stats_out[:, 1]
    idx32 = idx_out[:, :K]
    idx_sc = (idx32.reshape(NC * NS, ROWS_PER_TILE, K)
              .transpose(0, 2, 1)
              .reshape(NC * NS, K * ROWS_PER_TILE))
    return _score_stage(kth, a_k, idx_sc)


# speedup vs baseline: 100.4439x; 1.0186x over previous
import functools

import jax
import jax.numpy as jnp
from jax import lax
from jax.experimental import pallas as pl
from jax.experimental.pallas import tpu as pltpu
from jax.experimental.pallas import tpu_sc as plsc

B = 8192
D_IN = 768
D = 256
K = 32
NSEL = K + 1

R = 256
CT = 512
CTB = 2048
T = 4
NC = 2
NS = 16
ROWS_PER_TILE = B // (NC * NS)

_BIG_I = 2**31 - 1
_INF = float("inf")


def _proj_body(img_ref, w_ref, out_ref):
    out_ref[...] = lax.dot_general(
        img_ref[...], w_ref[...], (((1,), (0,)), ((), ())),
        precision=lax.Precision.HIGHEST,
        preferred_element_type=jnp.float32)


def _project(images, w):
    rb = 512
    return pl.pallas_call(
        _proj_body,
        grid=(B // rb,),
        in_specs=[
            pl.BlockSpec((rb, D_IN), lambda i: (i, 0)),
            pl.BlockSpec((D_IN, D), lambda i: (0, 0)),
        ],
        out_specs=pl.BlockSpec((rb, D), lambda i: (i, 0)),
        out_shape=jax.ShapeDtypeStruct((B, D), jnp.float32),
    )(images, w)


def _topk_body(xr_ref, vf_ref, idx_ref, stats_ref,
               candk_ref, x2row_ref, ko_ref, vd_ref):
    i = pl.program_id(0)

    @pl.when(i == 0)
    def _():
        vff = vf_ref[...].astype(jnp.float32)
        vf2 = vff * vff
        x2row_ref[...] = lax.dot_general(
            jnp.ones((8, D), jnp.float32), vf2, (((1,), (1,)), ((), ())),
            precision=lax.Precision.HIGHEST,
            preferred_element_type=jnp.float32)

    candk_ref[...] = jnp.full((R, T * CT), _BIG_I, jnp.int32)

    x_rb = xr_ref[...]
    xf = x_rb.astype(jnp.float32)
    x2_rb = jnp.sum(xf * xf, axis=1, keepdims=True)
    grow = (lax.broadcasted_iota(jnp.int32, (R, CTB), 0) + i * R)
    gcol_base = lax.broadcasted_iota(jnp.int32, (R, CTB), 1)

    def _ce(a, b):
        return jnp.minimum(a, b), jnp.maximum(a, b)

    def _sort4(s):
        s[0], s[1] = _ce(s[0], s[1])
        s[2], s[3] = _ce(s[2], s[3])
        s[0], s[2] = _ce(s[0], s[2])
        s[1], s[3] = _ce(s[1], s[3])
        s[1], s[2] = _ce(s[1], s[2])
        return s

    def _merge2(A, Bl):
        lo = [jnp.minimum(A[k], Bl[T - 1 - k]) for k in range(T)]
        lo[0], lo[2] = _ce(lo[0], lo[2])
        lo[1], lo[3] = _ce(lo[1], lo[3])
        lo[0], lo[1] = _ce(lo[0], lo[1])
        lo[2], lo[3] = _ce(lo[2], lo[3])
        return lo

    for ct in range(B // CTB):
        vfc = vf_ref[pl.ds(ct * CTB, CTB), :]
        dots = lax.dot_general(
            x_rb, vfc, (((1,), (1,)), ((), ())),
            preferred_element_type=jnp.float32)
        x2c = x2row_ref[0:1, pl.ds(ct * CTB, CTB)]
        d2 = x2_rb + x2c - 2.0 * dots
        gcol = gcol_base + ct * CTB
        diag = gcol == grow
        bits = lax.bitcast_convert_type(jnp.maximum(d2, 0.0), jnp.int32)
        keys = (bits & ~0x1FFF) | gcol
        keys = jnp.where(diag, _BIG_I, keys)

        @pl.when(i // (CTB // R) == ct)
        def _():
            vd_ref[...] = jnp.sum(jnp.where(diag, d2, 0.0), axis=1,
                                  keepdims=True)

        s = _sort4([keys[:, q * CT:(q + 1) * CT] for q in range(4)])
        lists = [candk_ref[:, t * CT:(t + 1) * CT] for t in range(T)]
        merged = _merge2(lists, s)
        for t in range(T):
            candk_ref[:, t * CT:(t + 1) * CT] = merged[t]

    v_d = vd_ref[...]

    cols = []
    for c in range(4):
        cols.append([candk_ref[:, t * CT + c * 128: t * CT + c * 128 + 128]
                     for t in range(T)])
    merged = _merge2(_merge2(cols[0], cols[1]), _merge2(cols[2], cols[3]))
    for t in range(T):
        candk_ref[:, t * CT: t * CT + 128] = merged[t]

    lane128 = lax.broadcasted_iota(jnp.int32, (R, 128), 1)
    rowid = (lax.broadcasted_iota(jnp.int32, (R, 1), 0) + i * R)

    def ex_body(k, _):
        top = candk_ref[:, 0:128]
        m = jnp.min(top, axis=1, keepdims=True)
        lane_mask = top == m
        for t in range(T - 1):
            sl = pl.ds(t * CT, 128)
            sn = pl.ds((t + 1) * CT, 128)
            candk_ref[:, sl] = jnp.where(lane_mask, candk_ref[:, sn],
                                         candk_ref[:, sl])
        sl = pl.ds((T - 1) * CT, 128)
        candk_ref[:, sl] = jnp.where(lane_mask, _BIG_I, candk_ref[:, sl])
        ko_ref[...] = jnp.where(lane128 == k, m, ko_ref[...])
        return 0

    lax.fori_loop(0, NSEL, ex_body, 0)

    ko = ko_ref[...]
    vals = lax.bitcast_convert_type(ko & ~0x1FFF, jnp.float32)
    g = ko & 0x1FFF
    idx_ref[...] = g - (g > rowid).astype(jnp.int32)

    v31 = vals[:, (K - 1):K]
    v32 = vals[:, K:(K + 1)]
    a_k = jnp.sqrt(jnp.maximum(v32, 0.0))
    kth2 = jnp.minimum(jnp.maximum(v_d, v31), v32)
    kth = jnp.sqrt(jnp.maximum(kth2, 0.0))
    stats_ref[...] = jnp.where(
        lane128 == 0, a_k, jnp.where(lane128 == 1, kth, 0.0))


def _topk(vf):
    vfb = vf.astype(jnp.bfloat16)
    return pl.pallas_call(
        _topk_body,
        grid=(B // R,),
        in_specs=[
            pl.BlockSpec((R, D), lambda i: (i, 0)),
            pl.BlockSpec((B, D), lambda i: (0, 0)),
        ],
        out_specs=[
            pl.BlockSpec((R, 128), lambda i: (i, 0)),
            pl.BlockSpec((R, 128), lambda i: (i, 0)),
        ],
        out_shape=[
            jax.ShapeDtypeStruct((B, 128), jnp.int32),
            jax.ShapeDtypeStruct((B, 128), jnp.float32),
        ],
        scratch_shapes=[
            pltpu.VMEM((R, T * CT), jnp.int32),
            pltpu.VMEM((8, B), jnp.float32),
            pltpu.VMEM((R, 128), jnp.int32),
            pltpu.VMEM((R, 1), jnp.float32),
        ],
    )(vfb, vfb)


def _sc_score_body(kth_hbm, ak_hbm, idx_hbm, out_hbm,
                   kth_v, ak_v, idx_v, sc_v):
    wid = lax.axis_index("s") * NC + lax.axis_index("c")
    base = wid * ROWS_PER_TILE
    pltpu.sync_copy(kth_hbm, kth_v)
    pltpu.sync_copy(ak_hbm.at[pl.ds(base, ROWS_PER_TILE)], ak_v)
    pltpu.sync_copy(idx_hbm.at[wid], idx_v)

    def g_body(g, _):
        r0 = g * 16
        ak16 = ak_v[pl.ds(r0, 16)]
        acc = jnp.zeros((16,), jnp.float32)
        for m in range(K):
            iv = idx_v[pl.ds(m * ROWS_PER_TILE + r0, 16)]
            dk = plsc.load_gather(kth_v, [iv])
            acc = acc + ak16 / dk
        s = acc * (1.0 / K)
        s = jnp.where(s != s, 1000.0, s)
        s = jnp.where(s == _INF, 1000.0, s)
        s = jnp.where(s == -_INF, 0.0, s)
        sc_v[pl.ds(r0, 16)] = s
        return 0

    lax.fori_loop(0, ROWS_PER_TILE // 16, g_body, 0)
    pltpu.sync_copy(sc_v, out_hbm.at[pl.ds(base, ROWS_PER_TILE)])


@functools.cache
def _make_sc_score():
    return pl.kernel(
        _sc_score_body,
        out_type=jax.ShapeDtypeStruct((B,), jnp.float32),
        mesh=plsc.VectorSubcoreMesh(core_axis_name="c", subcore_axis_name="s",
                                    num_cores=NC, num_subcores=NS),
        compiler_params=pltpu.CompilerParams(needs_layout_passes=False),
        scratch_types=[
            pltpu.VMEM((B,), jnp.float32),
            pltpu.VMEM((ROWS_PER_TILE,), jnp.float32),
            pltpu.VMEM((K * ROWS_PER_TILE,), jnp.int32),
            pltpu.VMEM((ROWS_PER_TILE,), jnp.float32),
        ],
    )


def _score_stage(kth, a_k, idx_sc):
    return _make_sc_score()(kth, a_k, idx_sc)


def kernel(images, W):
    vf = _project(images, W)
    idx_out, stats_out = _topk(vf)
    a_k = stats_out[:, 0]
    kth = stats_out[:, 1]
    idx32 = idx_out[:, :K]
    idx_sc = (idx32.reshape(NC * NS, ROWS_PER_TILE, K)
              .transpose(0, 2, 1)
              .reshape(NC * NS, K * ROWS_PER_TILE))
    return _score_stage(kth, a_k, idx_sc)
